# baseline (speedup 1.0000x reference)
import jax
import jax.numpy as jnp
from jax.experimental import pallas as pl

_N = 10000
_E = 320000
_NP = 10240


def _layer_mm(hcat, W, b, act):
    NP, K = hcat.shape
    dout = W.shape[1]
    BR = 1024

    def body(x_ref, w_ref, b_ref, o_ref):
        acc = jnp.dot(x_ref[...], w_ref[...], preferred_element_type=jnp.float32)
        acc = acc + b_ref[...]
        if act:
            acc = jnp.maximum(acc, 0.0)
        o_ref[...] = acc

    return pl.pallas_call(
        body,
        grid=(NP // BR,),
        in_specs=[
            pl.BlockSpec((BR, K), lambda i: (i, 0)),
            pl.BlockSpec((K, dout), lambda i: (0, 0)),
            pl.BlockSpec((1, dout), lambda i: (0, 0)),
        ],
        out_specs=pl.BlockSpec((BR, dout), lambda i: (i, 0)),
        out_shape=jax.ShapeDtypeStruct((NP, dout), jnp.float32),
    )(hcat, W, b.reshape(1, dout))


def kernel(features, edge_index, W0, b0, W1, b1, W2, b2):
    src = edge_index[0]
    dst = edge_index[1]
    deg = jax.ops.segment_sum(jnp.ones((_E,), jnp.float32), dst, num_segments=_N)
    deg = jnp.clip(deg, 1.0, None)
    norm = jax.lax.rsqrt(deg)

    def prop(h):
        h = h * norm[:, None]
        msg = jnp.take(h, src, axis=0)
        s = jax.ops.segment_sum(msg, dst, num_segments=_N)
        return s * norm[:, None]

    def layer(h, W, b, act):
        h1 = prop(h)
        h2 = prop(h1)
        hcat = jnp.concatenate([h, h1, h2], axis=-1)
        hcat = jnp.pad(hcat, ((0, _NP - _N), (0, 0)))
        out = _layer_mm(hcat, W, b, act)
        return out[:_N]

    h = layer(features, W0, b0, True)
    h = layer(h, W1, b1, True)
    h = layer(h, W2, b2, False)
    return h


# speedup vs baseline: 2.8230x; 2.8230x over previous
import functools

import jax
import jax.numpy as jnp
from jax import lax
from jax.experimental import pallas as pl
from jax.experimental.pallas import tpu as pltpu
from jax.experimental.pallas import tpu_sc as plsc

_N = 10000
_NP = 10240
_E = 320000
_EP = 327680
_B = 128
_NBC = 20
_RPT = _NP // 16

_f32 = jnp.float32


def _zero_rows(ref, nrows, ncols):
    zer = jnp.zeros((16,), _f32)

    def row(i, _):
        for k in range(ncols // 16):
            ref[i, pl.ds(k * 16, 16)] = zer
        return 0

    lax.fori_loop(0, nrows, row, 0, unroll=4)


def _make_deg_kernel():
    nb = _NBC
    mesh = plsc.VectorSubcoreMesh(core_axis_name="c", subcore_axis_name="s")

    @functools.partial(
        pl.kernel,
        out_type=(
            jax.ShapeDtypeStruct((_NP, 16), _f32),
            jax.ShapeDtypeStruct((_NP, 16), _f32),
        ),
        mesh=mesh,
        scratch_types=[
            pltpu.VMEM((nb, _B), jnp.int32),
            pltpu.VMEM((_B, 16), _f32),
            pltpu.VMEM((_RPT, 16), _f32),
            pltpu.VMEM_SHARED((_NP, 16), _f32),
        ],
    )
    def deg_kernel(dst_hbm, deg0, deg1, dstv, ones, stage, acc):
        c = lax.axis_index("c")
        s = lax.axis_index("s")
        row0 = s * _RPT
        w = c * 16 + s

        _zero_rows(stage, _RPT, 16)
        pltpu.sync_copy(stage, acc.at[pl.ds(row0, _RPT)])
        one = jnp.ones((16,), _f32)

        def orow(i, _):
            ones[i, :] = one
            return 0

        lax.fori_loop(0, _B, orow, 0, unroll=4)
        plsc.subcore_barrier()

        def step(j, _):
            pltpu.sync_copy(ones, acc.at[dstv.at[j]], add=True)
            return 0

        for k in range(4):
            pltpu.sync_copy(dst_hbm.at[4 * w + k], dstv)
            lax.fori_loop(0, nb, step, 0)
        plsc.subcore_barrier()
        pltpu.sync_copy(acc.at[pl.ds(row0, _RPT)], stage)

        @pl.when(c == 0)
        def _():
            pltpu.sync_copy(stage, deg0.at[pl.ds(row0, _RPT)])

        @pl.when(c == 1)
        def _():
            pltpu.sync_copy(stage, deg1.at[pl.ds(row0, _RPT)])

    return deg_kernel


def _edge_pipeline(yref, acc, srcv, dstv, rows0, rows1, gs0, gs1, nb):
    def step(j, _):
        pltpu.async_copy(yref.at[srcv.at[j]], rows0, gs0).wait()
        pltpu.sync_copy(rows0, acc.at[dstv.at[j]], add=True)
        return 0

    lax.fori_loop(0, nb, step, 0)


def _prop_scratch():
    return [
        pltpu.VMEM((_NBC, _B), jnp.int32),
        pltpu.VMEM((_NBC, _B), jnp.int32),
        pltpu.VMEM((_B, 128), _f32),
        pltpu.VMEM((_B, 128), _f32),
        pltpu.VMEM_SHARED((_NP, 128), _f32),
        pltpu.SemaphoreType.DMA,
        pltpu.SemaphoreType.DMA,
    ]


def _zero_acc(acc, rows0, row0):
    _zero_rows(rows0, _B, 128)
    for i in range(_RPT // _B):
        pltpu.sync_copy(rows0, acc.at[pl.ds(row0 + i * _B, _B)])


def _writeback(acc, rows0, row0, zref):
    for i in range(_RPT // _B):
        pltpu.sync_copy(acc.at[pl.ds(row0 + i * _B, _B)], rows0)
        pltpu.sync_copy(rows0, zref.at[pl.ds(row0 + i * _B, _B)])


def _chunked_pipeline(yref, acc, src_hbm, dst_hbm, srcv, dstv,
                      rows0, rows1, gs0, gs1, m0, nchunks):
    for k in range(nchunks):
        pltpu.sync_copy(src_hbm.at[m0 + k], srcv)
        pltpu.sync_copy(dst_hbm.at[m0 + k], dstv)
        _edge_pipeline(yref, acc, srcv, dstv, rows0, rows1, gs0, gs1, _NBC)


def _make_prop_edgesplit():
    mesh = plsc.VectorSubcoreMesh(core_axis_name="c", subcore_axis_name="s")

    @functools.partial(
        pl.kernel,
        out_type=(
            jax.ShapeDtypeStruct((_NP, 128), _f32),
            jax.ShapeDtypeStruct((_NP, 128), _f32),
        ),
        mesh=mesh,
        scratch_types=_prop_scratch(),
    )
    def prop_kernel(y, src_hbm, dst_hbm, z0, z1,
                    srcv, dstv, rows0, rows1, acc, gs0, gs1):
        c = lax.axis_index("c")
        s = lax.axis_index("s")
        row0 = s * _RPT
        w = c * 16 + s

        _zero_acc(acc, rows0, row0)
        plsc.subcore_barrier()

        _chunked_pipeline(y, acc, src_hbm, dst_hbm, srcv, dstv,
                          rows0, rows1, gs0, gs1, 4 * w, 4)
        plsc.subcore_barrier()

        @pl.when(c == 0)
        def _():
            _writeback(acc, rows0, row0, z0)

        @pl.when(c == 1)
        def _():
            _writeback(acc, rows0, row0, z1)

    return prop_kernel


def _make_prop_dimsplit():
    mesh = plsc.VectorSubcoreMesh(core_axis_name="c", subcore_axis_name="s")

    @functools.partial(
        pl.kernel,
        out_type=(
            jax.ShapeDtypeStruct((_NP, 128), _f32),
            jax.ShapeDtypeStruct((_NP, 128), _f32),
        ),
        mesh=mesh,
        scratch_types=_prop_scratch(),
    )
    def prop_kernel(y0, y1, src_hbm, dst_hbm, z0, z1,
                    srcv, dstv, rows0, rows1, acc, gs0, gs1):
        c = lax.axis_index("c")
        s = lax.axis_index("s")
        row0 = s * _RPT

        _zero_acc(acc, rows0, row0)
        plsc.subcore_barrier()

        def go(yref, zref):
            _chunked_pipeline(yref, acc, src_hbm, dst_hbm, srcv, dstv,
                              rows0, rows1, gs0, gs1, 8 * s, 8)
            plsc.subcore_barrier()
            _writeback(acc, rows0, row0, zref)

        @pl.when(c == 0)
        def _():
            go(y0, z0)

        @pl.when(c == 1)
        def _():
            go(y1, z1)

    return prop_kernel


def _norm_scale(deg0, deg1, x):
    def body(d0_ref, d1_ref, x_ref, norm_ref, y_ref):
        deg = d0_ref[:, :1] + d1_ref[:, :1]
        norm = lax.rsqrt(jnp.maximum(deg, 1.0))
        norm_ref[...] = norm
        y_ref[...] = x_ref[...] * norm

    return pl.pallas_call(
        body,
        out_shape=(
            jax.ShapeDtypeStruct((_NP, 1), _f32),
            jax.ShapeDtypeStruct((_NP, 128), _f32),
        ),
    )(deg0, deg1, x)


def _rescale_es(z0, z1, norm):
    def body(z0_ref, z1_ref, n_ref, y_ref):
        n2 = n_ref[...] * n_ref[...]
        y_ref[...] = (z0_ref[...] + z1_ref[...]) * n2

    return pl.pallas_call(
        body,
        out_shape=jax.ShapeDtypeStruct(z0.shape, _f32),
    )(z0, z1, norm)


def _rescale_ds(z0, z1, norm):
    def body(z0_ref, z1_ref, n_ref, y0_ref, y1_ref):
        n2 = n_ref[...] * n_ref[...]
        y0_ref[...] = z0_ref[...] * n2
        y1_ref[...] = z1_ref[...] * n2

    return pl.pallas_call(
        body,
        out_shape=(
            jax.ShapeDtypeStruct(z0.shape, _f32),
            jax.ShapeDtypeStruct(z1.shape, _f32),
        ),
    )(z0, z1, norm)


def _layer0_mm(x, z1p, z2p, norm, W, b):
    BR = 1024

    def body(x_ref, z1a_ref, z1b_ref, z2a_ref, z2b_ref, n_ref, w_ref, b_ref,
             out_ref, y0_ref, y1_ref):
        n = n_ref[...]
        hop1 = (z1a_ref[...] + z1b_ref[...]) * n
        hop2 = (z2a_ref[...] + z2b_ref[...]) * n
        acc = jnp.dot(x_ref[...], w_ref[:128, :], preferred_element_type=_f32)
        acc += jnp.dot(hop1, w_ref[128:256, :], preferred_element_type=_f32)
        acc += jnp.dot(hop2, w_ref[256:, :], preferred_element_type=_f32)
        acc += b_ref[...]
        acc = jnp.maximum(acc, 0.0)
        out_ref[...] = acc
        y = acc * n
        y0_ref[...] = y[:, :128]
        y1_ref[...] = y[:, 128:]

    blk = lambda cols: pl.BlockSpec((BR, cols), lambda i: (i, 0))
    return pl.pallas_call(
        body,
        grid=(_NP // BR,),
        in_specs=[
            blk(128), blk(128), blk(128), blk(128), blk(128), blk(1),
            pl.BlockSpec((384, 256), lambda i: (0, 0)),
            pl.BlockSpec((1, 256), lambda i: (0, 0)),
        ],
        out_specs=(blk(256), blk(128), blk(128)),
        out_shape=(
            jax.ShapeDtypeStruct((_NP, 256), _f32),
            jax.ShapeDtypeStruct((_NP, 128), _f32),
            jax.ShapeDtypeStruct((_NP, 128), _f32),
        ),
    )(x, z1p[0], z1p[1], z2p[0], z2p[1], norm, W, b.reshape(1, 256))


def _layer1_mm(h, z1p, z2p, norm, W, b, Wpq):
    BR = 1024

    def body(h_ref, z1a_ref, z1b_ref, z2a_ref, z2b_ref, n_ref, w_ref, b_ref,
             wpq_ref, out_ref, p_ref, r0_ref):
        n = n_ref[...]
        hop1 = jnp.concatenate([z1a_ref[...], z1b_ref[...]], axis=1) * n
        hop2 = jnp.concatenate([z2a_ref[...], z2b_ref[...]], axis=1) * n
        acc = jnp.dot(h_ref[...], w_ref[:256, :], preferred_element_type=_f32)
        acc += jnp.dot(hop1, w_ref[256:512, :], preferred_element_type=_f32)
        acc += jnp.dot(hop2, w_ref[512:, :], preferred_element_type=_f32)
        acc += b_ref[...]
        acc = jnp.maximum(acc, 0.0)
        out_ref[...] = acc
        proj = jnp.dot(acc, wpq_ref[...], preferred_element_type=_f32)
        p_ref[...] = proj[:, :16]
        r0 = proj[:, 16:] * n
        r0_ref[...] = jnp.concatenate(
            [r0, jnp.zeros((BR, 112), _f32)], axis=1)

    blk = lambda cols: pl.BlockSpec((BR, cols), lambda i: (i, 0))
    return pl.pallas_call(
        body,
        grid=(_NP // BR,),
        in_specs=[
            blk(256), blk(128), blk(128), blk(128), blk(128), blk(1),
            pl.BlockSpec((768, 256), lambda i: (0, 0)),
            pl.BlockSpec((1, 256), lambda i: (0, 0)),
            pl.BlockSpec((256, 32), lambda i: (0, 0)),
        ],
        out_specs=(blk(256), blk(16), blk(128)),
        out_shape=(
            jax.ShapeDtypeStruct((_NP, 256), _f32),
            jax.ShapeDtypeStruct((_NP, 16), _f32),
            jax.ShapeDtypeStruct((_NP, 128), _f32),
        ),
    )(h, z1p[0], z1p[1], z2p[0], z2p[1], norm, W, b.reshape(1, 256), Wpq)


def _mid16(p, t0, t1, norm):
    def body(p_ref, t0_ref, t1_ref, n_ref, r_ref):
        n = n_ref[...]
        t = (t0_ref[:, :16] + t1_ref[:, :16]) * (n * n)
        r = p_ref[...] * n + t
        r_ref[...] = jnp.concatenate(
            [r, jnp.zeros((_NP, 112), _f32)], axis=1)

    return pl.pallas_call(
        body,
        out_shape=jax.ShapeDtypeStruct((_NP, 128), _f32),
    )(p, t0, t1, norm)


def _final(h2, W2a, b2, u0, u1, norm):
    BR = 2048

    def body(h_ref, w_ref, b_ref, u0_ref, u1_ref, n_ref, o_ref):
        acc = jnp.dot(h_ref[...], w_ref[...], preferred_element_type=_f32)
        u = (u0_ref[:, :16] + u1_ref[:, :16]) * n_ref[...]
        o_ref[...] = acc + b_ref[...] + u

    return pl.pallas_call(
        body,
        grid=(_NP // BR,),
        in_specs=[
            pl.BlockSpec((BR, 256), lambda i: (i, 0)),
            pl.BlockSpec((256, 16), lambda i: (0, 0)),
            pl.BlockSpec((1, 16), lambda i: (0, 0)),
            pl.BlockSpec((BR, 128), lambda i: (i, 0)),
            pl.BlockSpec((BR, 128), lambda i: (i, 0)),
            pl.BlockSpec((BR, 1), lambda i: (i, 0)),
        ],
        out_specs=pl.BlockSpec((BR, 16), lambda i: (i, 0)),
        out_shape=jax.ShapeDtypeStruct((_NP, 16), _f32),
    )(h2, W2a, b2.reshape(1, 16), u0, u1, norm)


def kernel(features, edge_index, W0, b0, W1, b1, W2, b2):
    src = jnp.concatenate(
        [edge_index[0], jnp.full((_EP - _E,), _NP - 1, jnp.int32)])
    dst = jnp.concatenate(
        [edge_index[1], jnp.full((_EP - _E,), _NP - 1, jnp.int32)])
    src_es = src.reshape(128, _NBC, _B)
    dst_es = dst.reshape(128, _NBC, _B)
    src_ds = src_es
    dst_ds = dst_es

    x = jnp.pad(features, ((0, _NP - _N), (0, 0)))

    degj = jax.ops.segment_sum(jnp.ones((_E,), _f32), edge_index[1],
                               num_segments=_NP).reshape(_NP, 1)
    deg0 = jnp.broadcast_to(degj, (_NP, 16))
    deg1 = jnp.zeros((_NP, 16), _f32)
    norm, y = _norm_scale(deg0, deg1, x)


    prop_es = _make_prop_edgesplit()
    prop_ds = _make_prop_dimsplit()

    z1a, z1b = prop_es(y, src_es, dst_es)
    yb = _rescale_es(z1a, z1b, norm)
    z2a, z2b = prop_es(yb, src_es, dst_es)
    h1, ya0, ya1 = _layer0_mm(x, (z1a, z1b), (z2a, z2b), norm, W0, b0)

    z10, z11 = prop_ds(ya0, ya1, src_ds, dst_ds)
    yb0, yb1 = _rescale_ds(z10, z11, norm)
    z20, z21 = prop_ds(yb0, yb1, src_ds, dst_ds)
    h2, p, r0 = _layer1_mm(h1, (z10, z11), (z20, z21), norm, W1, b1,
                           jnp.concatenate([W2[256:512], W2[512:768]], axis=1))

    t0, t1 = prop_es(r0, src_es, dst_es)
    r1 = _mid16(p, t0, t1, norm)
    u0, u1 = prop_es(r1, src_es, dst_es)
    out = _final(h2, W2[:256], b2, u0, u1, norm)
    return out[:_N]


# speedup vs baseline: 2.9878x; 1.0584x over previous
import functools

import jax
import jax.numpy as jnp
from jax import lax
from jax.experimental import pallas as pl
from jax.experimental.pallas import tpu as pltpu
from jax.experimental.pallas import tpu_sc as plsc

_N = 10000
_NP = 10240
_E = 320000
_EP = 327680
_B = 128
_NBC = 20
_RPT = _NP // 16

_f32 = jnp.float32


def _zero_rows(ref, nrows, ncols):
    zer = jnp.zeros((16,), _f32)

    def row(i, _):
        for k in range(ncols // 16):
            ref[i, pl.ds(k * 16, 16)] = zer
        return 0

    lax.fori_loop(0, nrows, row, 0, unroll=4)


def _make_deg_kernel():
    mesh = plsc.VectorSubcoreMesh(core_axis_name="c", subcore_axis_name="s")

    @functools.partial(
        pl.kernel,
        out_type=(
            jax.ShapeDtypeStruct((_NP, 128), _f32),
            jax.ShapeDtypeStruct((_NP, 128), _f32),
        ),
        mesh=mesh,
        scratch_types=[
            pltpu.VMEM((_NBC, _B), jnp.int32),
            pltpu.VMEM((_B, 128), _f32),
            pltpu.VMEM_SHARED((_NP, 128), _f32),
            pltpu.SemaphoreType.DMA,
            pltpu.SemaphoreType.DMA,
        ],
    )
    def deg_kernel(dst_hbm, deg0, deg1, dstv, ones, acc, ss0, ss1):
        c = lax.axis_index("c")
        s = lax.axis_index("s")
        row0 = s * _RPT
        w = c * 16 + s

        _zero_acc(acc, ones, row0)
        one = jnp.ones((16,), _f32)

        def orow(i, _):
            for k in range(8):
                ones[i, pl.ds(k * 16, 16)] = one
            return 0

        lax.fori_loop(0, _B, orow, 0, unroll=4)
        plsc.subcore_barrier()

        def step(g, _):
            j0 = g * 2
            s0 = pltpu.async_copy(ones, acc.at[dstv.at[j0]], ss0, add=True)
            s1 = pltpu.async_copy(ones, acc.at[dstv.at[j0 + 1]], ss1, add=True)
            s0.wait()
            s1.wait()
            return 0

        for k in range(4):
            pltpu.sync_copy(dst_hbm.at[4 * w + k], dstv)
            lax.fori_loop(0, _NBC // 2, step, 0)
        plsc.subcore_barrier()

        @pl.when(c == 0)
        def _():
            _writeback(acc, ones, row0, deg0)

        @pl.when(c == 1)
        def _():
            _writeback(acc, ones, row0, deg1)

    return deg_kernel


def _edge_pipeline(yref, acc, srcv, dstv, rows0, rows1, gs0, gs1,
                   ss0, ss1, nb):
    def step(g, _):
        j0 = g * 2
        d0 = pltpu.async_copy(yref.at[srcv.at[j0]], rows0, gs0)
        d1 = pltpu.async_copy(yref.at[srcv.at[j0 + 1]], rows1, gs1)
        d0.wait()
        s0 = pltpu.async_copy(rows0, acc.at[dstv.at[j0]], ss0, add=True)
        d1.wait()
        s1 = pltpu.async_copy(rows1, acc.at[dstv.at[j0 + 1]], ss1, add=True)
        s0.wait()
        s1.wait()
        return 0

    lax.fori_loop(0, nb // 2, step, 0)


def _prop_scratch():
    return [
        pltpu.VMEM((_NBC, _B), jnp.int32),
        pltpu.VMEM((_NBC, _B), jnp.int32),
        pltpu.VMEM((_B, 128), _f32),
        pltpu.VMEM((_B, 128), _f32),
        pltpu.VMEM_SHARED((_NP, 128), _f32),
        pltpu.SemaphoreType.DMA,
        pltpu.SemaphoreType.DMA,
        pltpu.SemaphoreType.DMA,
        pltpu.SemaphoreType.DMA,
    ]


def _zero_acc(acc, rows0, row0):
    _zero_rows(rows0, _B, 128)
    for i in range(_RPT // _B):
        pltpu.sync_copy(rows0, acc.at[pl.ds(row0 + i * _B, _B)])


def _writeback(acc, rows0, row0, zref):
    for i in range(_RPT // _B):
        pltpu.sync_copy(acc.at[pl.ds(row0 + i * _B, _B)], rows0)
        pltpu.sync_copy(rows0, zref.at[pl.ds(row0 + i * _B, _B)])


def _chunked_pipeline(yref, acc, src_hbm, dst_hbm, srcv, dstv,
                      rows0, rows1, gs0, gs1, ss0, ss1, m0, nchunks):
    for k in range(nchunks):
        pltpu.sync_copy(src_hbm.at[m0 + k], srcv)
        pltpu.sync_copy(dst_hbm.at[m0 + k], dstv)
        _edge_pipeline(yref, acc, srcv, dstv, rows0, rows1, gs0, gs1,
                       ss0, ss1, _NBC)


def _make_prop_edgesplit():
    mesh = plsc.VectorSubcoreMesh(core_axis_name="c", subcore_axis_name="s")

    @functools.partial(
        pl.kernel,
        out_type=(
            jax.ShapeDtypeStruct((_NP, 128), _f32),
            jax.ShapeDtypeStruct((_NP, 128), _f32),
        ),
        mesh=mesh,
        scratch_types=_prop_scratch(),
    )
    def prop_kernel(y, src_hbm, dst_hbm, z0, z1,
                    srcv, dstv, rows0, rows1, acc, gs0, gs1, ss0, ss1):
        c = lax.axis_index("c")
        s = lax.axis_index("s")
        row0 = s * _RPT
        w = c * 16 + s

        _zero_acc(acc, rows0, row0)
        plsc.subcore_barrier()

        _chunked_pipeline(y, acc, src_hbm, dst_hbm, srcv, dstv,
                          rows0, rows1, gs0, gs1, ss0, ss1, 4 * w, 4)
        plsc.subcore_barrier()

        @pl.when(c == 0)
        def _():
            _writeback(acc, rows0, row0, z0)

        @pl.when(c == 1)
        def _():
            _writeback(acc, rows0, row0, z1)

    return prop_kernel


def _make_prop_dimsplit():
    mesh = plsc.VectorSubcoreMesh(core_axis_name="c", subcore_axis_name="s")

    @functools.partial(
        pl.kernel,
        out_type=(
            jax.ShapeDtypeStruct((_NP, 128), _f32),
            jax.ShapeDtypeStruct((_NP, 128), _f32),
        ),
        mesh=mesh,
        scratch_types=_prop_scratch(),
    )
    def prop_kernel(y0, y1, src_hbm, dst_hbm, z0, z1,
                    srcv, dstv, rows0, rows1, acc, gs0, gs1, ss0, ss1):
        c = lax.axis_index("c")
        s = lax.axis_index("s")
        row0 = s * _RPT

        _zero_acc(acc, rows0, row0)
        plsc.subcore_barrier()

        def go(yref, zref):
            _chunked_pipeline(yref, acc, src_hbm, dst_hbm, srcv, dstv,
                              rows0, rows1, gs0, gs1, ss0, ss1, 8 * s, 8)
            plsc.subcore_barrier()
            _writeback(acc, rows0, row0, zref)

        @pl.when(c == 0)
        def _():
            go(y0, z0)

        @pl.when(c == 1)
        def _():
            go(y1, z1)

    return prop_kernel


def _norm_scale(deg0, deg1, x):
    def body(d0_ref, d1_ref, x_ref, norm_ref, y_ref):
        deg = d0_ref[:, :1] + d1_ref[:, :1]
        norm = lax.rsqrt(jnp.maximum(deg, 1.0))
        norm_ref[...] = norm
        y_ref[...] = x_ref[...] * norm

    return pl.pallas_call(
        body,
        out_shape=(
            jax.ShapeDtypeStruct((_NP, 1), _f32),
            jax.ShapeDtypeStruct((_NP, 128), _f32),
        ),
    )(deg0, deg1, x)


def _rescale_es(z0, z1, norm):
    def body(z0_ref, z1_ref, n_ref, y_ref):
        n2 = n_ref[...] * n_ref[...]
        y_ref[...] = (z0_ref[...] + z1_ref[...]) * n2

    return pl.pallas_call(
        body,
        out_shape=jax.ShapeDtypeStruct(z0.shape, _f32),
    )(z0, z1, norm)


def _rescale_ds(z0, z1, norm):
    def body(z0_ref, z1_ref, n_ref, y0_ref, y1_ref):
        n2 = n_ref[...] * n_ref[...]
        y0_ref[...] = z0_ref[...] * n2
        y1_ref[...] = z1_ref[...] * n2

    return pl.pallas_call(
        body,
        out_shape=(
            jax.ShapeDtypeStruct(z0.shape, _f32),
            jax.ShapeDtypeStruct(z1.shape, _f32),
        ),
    )(z0, z1, norm)


def _layer0_mm(x, z1p, z2p, norm, W, b):
    BR = 1024

    def body(x_ref, z1a_ref, z1b_ref, z2a_ref, z2b_ref, n_ref, w_ref, b_ref,
             out_ref, y0_ref, y1_ref):
        n = n_ref[...]
        hop1 = (z1a_ref[...] + z1b_ref[...]) * n
        hop2 = (z2a_ref[...] + z2b_ref[...]) * n
        acc = jnp.dot(x_ref[...], w_ref[:128, :], preferred_element_type=_f32)
        acc += jnp.dot(hop1, w_ref[128:256, :], preferred_element_type=_f32)
        acc += jnp.dot(hop2, w_ref[256:, :], preferred_element_type=_f32)
        acc += b_ref[...]
        acc = jnp.maximum(acc, 0.0)
        out_ref[...] = acc
        y = acc * n
        y0_ref[...] = y[:, :128]
        y1_ref[...] = y[:, 128:]

    blk = lambda cols: pl.BlockSpec((BR, cols), lambda i: (i, 0))
    return pl.pallas_call(
        body,
        grid=(_NP // BR,),
        in_specs=[
            blk(128), blk(128), blk(128), blk(128), blk(128), blk(1),
            pl.BlockSpec((384, 256), lambda i: (0, 0)),
            pl.BlockSpec((1, 256), lambda i: (0, 0)),
        ],
        out_specs=(blk(256), blk(128), blk(128)),
        out_shape=(
            jax.ShapeDtypeStruct((_NP, 256), _f32),
            jax.ShapeDtypeStruct((_NP, 128), _f32),
            jax.ShapeDtypeStruct((_NP, 128), _f32),
        ),
    )(x, z1p[0], z1p[1], z2p[0], z2p[1], norm, W, b.reshape(1, 256))


def _layer1_mm(h, z1p, z2p, norm, W, b, Wpq):
    BR = 1024

    def body(h_ref, z1a_ref, z1b_ref, z2a_ref, z2b_ref, n_ref, w_ref, b_ref,
             wpq_ref, out_ref, p_ref, r0_ref):
        n = n_ref[...]
        hop1 = jnp.concatenate([z1a_ref[...], z1b_ref[...]], axis=1) * n
        hop2 = jnp.concatenate([z2a_ref[...], z2b_ref[...]], axis=1) * n
        acc = jnp.dot(h_ref[...], w_ref[:256, :], preferred_element_type=_f32)
        acc += jnp.dot(hop1, w_ref[256:512, :], preferred_element_type=_f32)
        acc += jnp.dot(hop2, w_ref[512:, :], preferred_element_type=_f32)
        acc += b_ref[...]
        acc = jnp.maximum(acc, 0.0)
        out_ref[...] = acc
        proj = jnp.dot(acc, wpq_ref[...], preferred_element_type=_f32)
        p_ref[...] = proj[:, :16]
        r0 = proj[:, 16:] * n
        r0_ref[...] = jnp.concatenate(
            [r0, jnp.zeros((BR, 112), _f32)], axis=1)

    blk = lambda cols: pl.BlockSpec((BR, cols), lambda i: (i, 0))
    return pl.pallas_call(
        body,
        grid=(_NP // BR,),
        in_specs=[
            blk(256), blk(128), blk(128), blk(128), blk(128), blk(1),
            pl.BlockSpec((768, 256), lambda i: (0, 0)),
            pl.BlockSpec((1, 256), lambda i: (0, 0)),
            pl.BlockSpec((256, 32), lambda i: (0, 0)),
        ],
        out_specs=(blk(256), blk(16), blk(128)),
        out_shape=(
            jax.ShapeDtypeStruct((_NP, 256), _f32),
            jax.ShapeDtypeStruct((_NP, 16), _f32),
            jax.ShapeDtypeStruct((_NP, 128), _f32),
        ),
    )(h, z1p[0], z1p[1], z2p[0], z2p[1], norm, W, b.reshape(1, 256), Wpq)


def _mid16(p, t0, t1, norm):
    def body(p_ref, t0_ref, t1_ref, n_ref, r_ref):
        n = n_ref[...]
        t = (t0_ref[:, :16] + t1_ref[:, :16]) * (n * n)
        r = p_ref[...] * n + t
        r_ref[...] = jnp.concatenate(
            [r, jnp.zeros((_NP, 112), _f32)], axis=1)

    return pl.pallas_call(
        body,
        out_shape=jax.ShapeDtypeStruct((_NP, 128), _f32),
    )(p, t0, t1, norm)


def _final(h2, W2a, b2, u0, u1, norm):
    BR = 2048

    def body(h_ref, w_ref, b_ref, u0_ref, u1_ref, n_ref, o_ref):
        acc = jnp.dot(h_ref[...], w_ref[...], preferred_element_type=_f32)
        u = (u0_ref[:, :16] + u1_ref[:, :16]) * n_ref[...]
        o_ref[...] = acc + b_ref[...] + u

    return pl.pallas_call(
        body,
        grid=(_NP // BR,),
        in_specs=[
            pl.BlockSpec((BR, 256), lambda i: (i, 0)),
            pl.BlockSpec((256, 16), lambda i: (0, 0)),
            pl.BlockSpec((1, 16), lambda i: (0, 0)),
            pl.BlockSpec((BR, 128), lambda i: (i, 0)),
            pl.BlockSpec((BR, 128), lambda i: (i, 0)),
            pl.BlockSpec((BR, 1), lambda i: (i, 0)),
        ],
        out_specs=pl.BlockSpec((BR, 16), lambda i: (i, 0)),
        out_shape=jax.ShapeDtypeStruct((_NP, 16), _f32),
    )(h2, W2a, b2.reshape(1, 16), u0, u1, norm)


def kernel(features, edge_index, W0, b0, W1, b1, W2, b2):
    src = jnp.concatenate(
        [edge_index[0], jnp.full((_EP - _E,), _NP - 1, jnp.int32)])
    dst = jnp.concatenate(
        [edge_index[1], jnp.full((_EP - _E,), _NP - 1, jnp.int32)])
    src_es = src.reshape(128, _NBC, _B)
    dst_es = dst.reshape(128, _NBC, _B)
    src_ds = src_es
    dst_ds = dst_es

    x = jnp.pad(features, ((0, _NP - _N), (0, 0)))

    deg0, deg1 = _make_deg_kernel()(dst_es)
    norm, y = _norm_scale(deg0, deg1, x)


    prop_es = _make_prop_edgesplit()
    prop_ds = _make_prop_dimsplit()

    z1a, z1b = prop_es(y, src_es, dst_es)
    yb = _rescale_es(z1a, z1b, norm)
    z2a, z2b = prop_es(yb, src_es, dst_es)
    h1, ya0, ya1 = _layer0_mm(x, (z1a, z1b), (z2a, z2b), norm, W0, b0)

    z10, z11 = prop_ds(ya0, ya1, src_ds, dst_ds)
    yb0, yb1 = _rescale_ds(z10, z11, norm)
    z20, z21 = prop_ds(yb0, yb1, src_ds, dst_ds)
    h2, p, r0 = _layer1_mm(h1, (z10, z11), (z20, z21), norm, W1, b1,
                           jnp.concatenate([W2[256:512], W2[512:768]], axis=1))

    t0, t1 = prop_es(r0, src_es, dst_es)
    r1 = _mid16(p, t0, t1, norm)
    u0, u1 = prop_es(r1, src_es, dst_es)
    out = _final(h2, W2[:256], b2, u0, u1, norm)
    return out[:_N]


# speedup vs baseline: 3.1204x; 1.0444x over previous
import functools

import jax
import jax.numpy as jnp
from jax import lax
from jax.experimental import pallas as pl
from jax.experimental.pallas import tpu as pltpu
from jax.experimental.pallas import tpu_sc as plsc

_N = 10000
_NP = 10240
_E = 320000
_EP = 327680
_B = 128
_NBC = 20
_RPT = _NP // 16

_f32 = jnp.float32


def _zero_rows(ref, nrows, ncols):
    zer = jnp.zeros((16,), _f32)

    def row(i, _):
        for k in range(ncols // 16):
            ref[i, pl.ds(k * 16, 16)] = zer
        return 0

    lax.fori_loop(0, nrows, row, 0, unroll=4)


def _make_deg_kernel():
    mesh = plsc.VectorSubcoreMesh(core_axis_name="c", subcore_axis_name="s")

    @functools.partial(
        pl.kernel,
        out_type=(
            jax.ShapeDtypeStruct((_NP, 128), _f32),
            jax.ShapeDtypeStruct((_NP, 128), _f32),
        ),
        mesh=mesh,
        scratch_types=[
            pltpu.VMEM((_NBC, _B), jnp.int32),
            pltpu.VMEM((_B, 128), _f32),
            pltpu.VMEM_SHARED((_NP, 128), _f32),
            pltpu.SemaphoreType.DMA,
            pltpu.SemaphoreType.DMA,
        ],
    )
    def deg_kernel(dst_hbm, deg0, deg1, dstv, ones, acc, ss0, ss1):
        c = lax.axis_index("c")
        s = lax.axis_index("s")
        row0 = s * _RPT
        w = c * 16 + s

        _zero_acc(acc, ones, row0)
        one = jnp.ones((16,), _f32)

        def orow(i, _):
            for k in range(8):
                ones[i, pl.ds(k * 16, 16)] = one
            return 0

        lax.fori_loop(0, _B, orow, 0, unroll=4)
        plsc.subcore_barrier()

        def step(g, _):
            j0 = g * 2
            s0 = pltpu.async_copy(ones, acc.at[dstv.at[j0]], ss0, add=True)
            s1 = pltpu.async_copy(ones, acc.at[dstv.at[j0 + 1]], ss1, add=True)
            s0.wait()
            s1.wait()
            return 0

        for k in range(4):
            pltpu.sync_copy(dst_hbm.at[4 * w + k], dstv)
            lax.fori_loop(0, _NBC // 2, step, 0)
        plsc.subcore_barrier()

        @pl.when(c == 0)
        def _():
            _writeback(acc, ones, row0, deg0)

        @pl.when(c == 1)
        def _():
            _writeback(acc, ones, row0, deg1)

    return deg_kernel


def _edge_pipeline(yref, acc, srcv, dstv, rows0, rows1, gs0, gs1,
                   ss0, ss1, nb):
    pltpu.async_copy(yref.at[srcv.at[0]], rows0, gs0)
    pltpu.async_copy(yref.at[srcv.at[1]], rows1, gs1)

    def step(g, _):
        j0 = g * 2
        pltpu.make_async_copy(yref.at[srcv.at[j0]], rows0, gs0).wait()
        s0 = pltpu.async_copy(rows0, acc.at[dstv.at[j0]], ss0, add=True)
        pltpu.make_async_copy(yref.at[srcv.at[j0 + 1]], rows1, gs1).wait()
        s1 = pltpu.async_copy(rows1, acc.at[dstv.at[j0 + 1]], ss1, add=True)
        s0.wait()

        @pl.when(j0 + 2 < nb)
        def _():
            pltpu.async_copy(yref.at[srcv.at[j0 + 2]], rows0, gs0)

        s1.wait()

        @pl.when(j0 + 3 < nb)
        def _():
            pltpu.async_copy(yref.at[srcv.at[j0 + 3]], rows1, gs1)

        return 0

    lax.fori_loop(0, nb // 2, step, 0)


def _prop_scratch():
    return [
        pltpu.VMEM((_NBC, _B), jnp.int32),
        pltpu.VMEM((_NBC, _B), jnp.int32),
        pltpu.VMEM((_B, 128), _f32),
        pltpu.VMEM((_B, 128), _f32),
        pltpu.VMEM_SHARED((_NP, 128), _f32),
        pltpu.SemaphoreType.DMA,
        pltpu.SemaphoreType.DMA,
        pltpu.SemaphoreType.DMA,
        pltpu.SemaphoreType.DMA,
    ]


def _zero_acc(acc, rows0, row0):
    _zero_rows(rows0, _B, 128)
    for i in range(_RPT // _B):
        pltpu.sync_copy(rows0, acc.at[pl.ds(row0 + i * _B, _B)])


def _writeback(acc, rows0, row0, zref):
    for i in range(_RPT // _B):
        pltpu.sync_copy(acc.at[pl.ds(row0 + i * _B, _B)], rows0)
        pltpu.sync_copy(rows0, zref.at[pl.ds(row0 + i * _B, _B)])


def _chunked_pipeline(yref, acc, src_hbm, dst_hbm, srcv, dstv,
                      rows0, rows1, gs0, gs1, ss0, ss1, m0, nchunks):
    for k in range(nchunks):
        pltpu.sync_copy(src_hbm.at[m0 + k], srcv)
        pltpu.sync_copy(dst_hbm.at[m0 + k], dstv)
        _edge_pipeline(yref, acc, srcv, dstv, rows0, rows1, gs0, gs1,
                       ss0, ss1, _NBC)


def _make_prop_edgesplit():
    mesh = plsc.VectorSubcoreMesh(core_axis_name="c", subcore_axis_name="s")

    @functools.partial(
        pl.kernel,
        out_type=(
            jax.ShapeDtypeStruct((_NP, 128), _f32),
            jax.ShapeDtypeStruct((_NP, 128), _f32),
        ),
        mesh=mesh,
        scratch_types=_prop_scratch(),
    )
    def prop_kernel(y, src_hbm, dst_hbm, z0, z1,
                    srcv, dstv, rows0, rows1, acc, gs0, gs1, ss0, ss1):
        c = lax.axis_index("c")
        s = lax.axis_index("s")
        row0 = s * _RPT
        w = c * 16 + s

        _zero_acc(acc, rows0, row0)
        plsc.subcore_barrier()

        _chunked_pipeline(y, acc, src_hbm, dst_hbm, srcv, dstv,
                          rows0, rows1, gs0, gs1, ss0, ss1, 4 * w, 4)
        plsc.subcore_barrier()

        @pl.when(c == 0)
        def _():
            _writeback(acc, rows0, row0, z0)

        @pl.when(c == 1)
        def _():
            _writeback(acc, rows0, row0, z1)

    return prop_kernel


def _make_prop_dimsplit():
    mesh = plsc.VectorSubcoreMesh(core_axis_name="c", subcore_axis_name="s")

    @functools.partial(
        pl.kernel,
        out_type=(
            jax.ShapeDtypeStruct((_NP, 128), _f32),
            jax.ShapeDtypeStruct((_NP, 128), _f32),
        ),
        mesh=mesh,
        scratch_types=_prop_scratch(),
    )
    def prop_kernel(y0, y1, src_hbm, dst_hbm, z0, z1,
                    srcv, dstv, rows0, rows1, acc, gs0, gs1, ss0, ss1):
        c = lax.axis_index("c")
        s = lax.axis_index("s")
        row0 = s * _RPT

        _zero_acc(acc, rows0, row0)
        plsc.subcore_barrier()

        def go(yref, zref):
            _chunked_pipeline(yref, acc, src_hbm, dst_hbm, srcv, dstv,
                              rows0, rows1, gs0, gs1, ss0, ss1, 8 * s, 8)
            plsc.subcore_barrier()
            _writeback(acc, rows0, row0, zref)

        @pl.when(c == 0)
        def _():
            go(y0, z0)

        @pl.when(c == 1)
        def _():
            go(y1, z1)

    return prop_kernel


def _norm_scale(deg0, deg1, x):
    def body(d0_ref, d1_ref, x_ref, norm_ref, y_ref):
        deg = d0_ref[:, :1] + d1_ref[:, :1]
        norm = lax.rsqrt(jnp.maximum(deg, 1.0))
        norm_ref[...] = norm
        y_ref[...] = x_ref[...] * norm

    return pl.pallas_call(
        body,
        out_shape=(
            jax.ShapeDtypeStruct((_NP, 1), _f32),
            jax.ShapeDtypeStruct((_NP, 128), _f32),
        ),
    )(deg0, deg1, x)


def _rescale_es(z0, z1, norm):
    def body(z0_ref, z1_ref, n_ref, y_ref):
        n2 = n_ref[...] * n_ref[...]
        y_ref[...] = (z0_ref[...] + z1_ref[...]) * n2

    return pl.pallas_call(
        body,
        out_shape=jax.ShapeDtypeStruct(z0.shape, _f32),
    )(z0, z1, norm)


def _rescale_ds(z0, z1, norm):
    def body(z0_ref, z1_ref, n_ref, y0_ref, y1_ref):
        n2 = n_ref[...] * n_ref[...]
        y0_ref[...] = z0_ref[...] * n2
        y1_ref[...] = z1_ref[...] * n2

    return pl.pallas_call(
        body,
        out_shape=(
            jax.ShapeDtypeStruct(z0.shape, _f32),
            jax.ShapeDtypeStruct(z1.shape, _f32),
        ),
    )(z0, z1, norm)


def _layer0_mm(x, z1p, z2p, norm, W, b):
    BR = 1024

    def body(x_ref, z1a_ref, z1b_ref, z2a_ref, z2b_ref, n_ref, w_ref, b_ref,
             out_ref, y0_ref, y1_ref):
        n = n_ref[...]
        hop1 = (z1a_ref[...] + z1b_ref[...]) * n
        hop2 = (z2a_ref[...] + z2b_ref[...]) * n
        acc = jnp.dot(x_ref[...], w_ref[:128, :], preferred_element_type=_f32)
        acc += jnp.dot(hop1, w_ref[128:256, :], preferred_element_type=_f32)
        acc += jnp.dot(hop2, w_ref[256:, :], preferred_element_type=_f32)
        acc += b_ref[...]
        acc = jnp.maximum(acc, 0.0)
        out_ref[...] = acc
        y = acc * n
        y0_ref[...] = y[:, :128]
        y1_ref[...] = y[:, 128:]

    blk = lambda cols: pl.BlockSpec((BR, cols), lambda i: (i, 0))
    return pl.pallas_call(
        body,
        grid=(_NP // BR,),
        in_specs=[
            blk(128), blk(128), blk(128), blk(128), blk(128), blk(1),
            pl.BlockSpec((384, 256), lambda i: (0, 0)),
            pl.BlockSpec((1, 256), lambda i: (0, 0)),
        ],
        out_specs=(blk(256), blk(128), blk(128)),
        out_shape=(
            jax.ShapeDtypeStruct((_NP, 256), _f32),
            jax.ShapeDtypeStruct((_NP, 128), _f32),
            jax.ShapeDtypeStruct((_NP, 128), _f32),
        ),
    )(x, z1p[0], z1p[1], z2p[0], z2p[1], norm, W, b.reshape(1, 256))


def _layer1_mm(h, z1p, z2p, norm, W, b, Wpq):
    BR = 1024

    def body(h_ref, z1a_ref, z1b_ref, z2a_ref, z2b_ref, n_ref, w_ref, b_ref,
             wpq_ref, out_ref, p_ref, r0_ref):
        n = n_ref[...]
        hop1 = jnp.concatenate([z1a_ref[...], z1b_ref[...]], axis=1) * n
        hop2 = jnp.concatenate([z2a_ref[...], z2b_ref[...]], axis=1) * n
        acc = jnp.dot(h_ref[...], w_ref[:256, :], preferred_element_type=_f32)
        acc += jnp.dot(hop1, w_ref[256:512, :], preferred_element_type=_f32)
        acc += jnp.dot(hop2, w_ref[512:, :], preferred_element_type=_f32)
        acc += b_ref[...]
        acc = jnp.maximum(acc, 0.0)
        out_ref[...] = acc
        proj = jnp.dot(acc, wpq_ref[...], preferred_element_type=_f32)
        p_ref[...] = proj[:, :16]
        r0 = proj[:, 16:] * n
        r0_ref[...] = jnp.concatenate(
            [r0, jnp.zeros((BR, 112), _f32)], axis=1)

    blk = lambda cols: pl.BlockSpec((BR, cols), lambda i: (i, 0))
    return pl.pallas_call(
        body,
        grid=(_NP // BR,),
        in_specs=[
            blk(256), blk(128), blk(128), blk(128), blk(128), blk(1),
            pl.BlockSpec((768, 256), lambda i: (0, 0)),
            pl.BlockSpec((1, 256), lambda i: (0, 0)),
            pl.BlockSpec((256, 32), lambda i: (0, 0)),
        ],
        out_specs=(blk(256), blk(16), blk(128)),
        out_shape=(
            jax.ShapeDtypeStruct((_NP, 256), _f32),
            jax.ShapeDtypeStruct((_NP, 16), _f32),
            jax.ShapeDtypeStruct((_NP, 128), _f32),
        ),
    )(h, z1p[0], z1p[1], z2p[0], z2p[1], norm, W, b.reshape(1, 256), Wpq)


def _mid16(p, t0, t1, norm):
    def body(p_ref, t0_ref, t1_ref, n_ref, r_ref):
        n = n_ref[...]
        t = (t0_ref[:, :16] + t1_ref[:, :16]) * (n * n)
        r = p_ref[...] * n + t
        r_ref[...] = jnp.concatenate(
            [r, jnp.zeros((_NP, 112), _f32)], axis=1)

    return pl.pallas_call(
        body,
        out_shape=jax.ShapeDtypeStruct((_NP, 128), _f32),
    )(p, t0, t1, norm)


def _final(h2, W2a, b2, u0, u1, norm):
    BR = 2048

    def body(h_ref, w_ref, b_ref, u0_ref, u1_ref, n_ref, o_ref):
        acc = jnp.dot(h_ref[...], w_ref[...], preferred_element_type=_f32)
        u = (u0_ref[:, :16] + u1_ref[:, :16]) * n_ref[...]
        o_ref[...] = acc + b_ref[...] + u

    return pl.pallas_call(
        body,
        grid=(_NP // BR,),
        in_specs=[
            pl.BlockSpec((BR, 256), lambda i: (i, 0)),
            pl.BlockSpec((256, 16), lambda i: (0, 0)),
            pl.BlockSpec((1, 16), lambda i: (0, 0)),
            pl.BlockSpec((BR, 128), lambda i: (i, 0)),
            pl.BlockSpec((BR, 128), lambda i: (i, 0)),
            pl.BlockSpec((BR, 1), lambda i: (i, 0)),
        ],
        out_specs=pl.BlockSpec((BR, 16), lambda i: (i, 0)),
        out_shape=jax.ShapeDtypeStruct((_NP, 16), _f32),
    )(h2, W2a, b2.reshape(1, 16), u0, u1, norm)


def kernel(features, edge_index, W0, b0, W1, b1, W2, b2):
    src = jnp.concatenate(
        [edge_index[0], jnp.full((_EP - _E,), _NP - 1, jnp.int32)])
    dst = jnp.concatenate(
        [edge_index[1], jnp.full((_EP - _E,), _NP - 1, jnp.int32)])
    src_es = src.reshape(128, _NBC, _B)
    dst_es = dst.reshape(128, _NBC, _B)
    src_ds = src_es
    dst_ds = dst_es

    x = jnp.pad(features, ((0, _NP - _N), (0, 0)))

    deg0, deg1 = _make_deg_kernel()(dst_es)
    norm, y = _norm_scale(deg0, deg1, x)


    prop_es = _make_prop_edgesplit()
    prop_ds = _make_prop_dimsplit()

    z1a, z1b = prop_es(y, src_es, dst_es)
    yb = _rescale_es(z1a, z1b, norm)
    z2a, z2b = prop_es(yb, src_es, dst_es)
    h1, ya0, ya1 = _layer0_mm(x, (z1a, z1b), (z2a, z2b), norm, W0, b0)

    z10, z11 = prop_ds(ya0, ya1, src_ds, dst_ds)
    yb0, yb1 = _rescale_ds(z10, z11, norm)
    z20, z21 = prop_ds(yb0, yb1, src_ds, dst_ds)
    h2, p, r0 = _layer1_mm(h1, (z10, z11), (z20, z21), norm, W1, b1,
                           jnp.concatenate([W2[256:512], W2[512:768]], axis=1))

    t0, t1 = prop_es(r0, src_es, dst_es)
    r1 = _mid16(p, t0, t1, norm)
    u0, u1 = prop_es(r1, src_es, dst_es)
    out = _final(h2, W2[:256], b2, u0, u1, norm)
    return out[:_N]


# speedup vs baseline: 3.1304x; 1.0032x over previous
import functools

import jax
import jax.numpy as jnp
from jax import lax
from jax.experimental import pallas as pl
from jax.experimental.pallas import tpu as pltpu
from jax.experimental.pallas import tpu_sc as plsc

_N = 10000
_NP = 10240
_E = 320000
_EP = 327680
_B = 128
_NBC = 20
_RPT = _NP // 16

_f32 = jnp.float32


def _zero_rows(ref, nrows, ncols):
    zer = jnp.zeros((16,), _f32)

    def row(i, _):
        for k in range(ncols // 16):
            ref[i, pl.ds(k * 16, 16)] = zer
        return 0

    lax.fori_loop(0, nrows, row, 0, unroll=4)


def _make_deg_kernel():
    mesh = plsc.VectorSubcoreMesh(core_axis_name="c", subcore_axis_name="s")

    @functools.partial(
        pl.kernel,
        out_type=(
            jax.ShapeDtypeStruct((_NP, 128), _f32),
            jax.ShapeDtypeStruct((_NP, 128), _f32),
        ),
        mesh=mesh,
        scratch_types=[
            pltpu.VMEM((_NBC, _B), jnp.int32),
            pltpu.VMEM((_B, 128), _f32),
            pltpu.VMEM_SHARED((_NP, 128), _f32),
            pltpu.SemaphoreType.DMA,
            pltpu.SemaphoreType.DMA,
        ],
    )
    def deg_kernel(dst_hbm, deg0, deg1, dstv, ones, acc, ss0, ss1):
        c = lax.axis_index("c")
        s = lax.axis_index("s")
        row0 = s * _RPT
        w = c * 16 + s

        _zero_acc(acc, ones, row0, ss0)
        one = jnp.ones((16,), _f32)

        def orow(i, _):
            for k in range(8):
                ones[i, pl.ds(k * 16, 16)] = one
            return 0

        lax.fori_loop(0, _B, orow, 0, unroll=4)
        plsc.subcore_barrier()

        def step(g, _):
            j0 = g * 2
            s0 = pltpu.async_copy(ones, acc.at[dstv.at[j0]], ss0, add=True)
            s1 = pltpu.async_copy(ones, acc.at[dstv.at[j0 + 1]], ss1, add=True)
            s0.wait()
            s1.wait()
            return 0

        for k in range(4):
            pltpu.sync_copy(dst_hbm.at[4 * w + k], dstv)
            lax.fori_loop(0, _NBC // 2, step, 0)
        plsc.subcore_barrier()

        def wb(zref):
            for i in range(_RPT // _B):
                pltpu.sync_copy(acc.at[pl.ds(row0 + i * _B, _B)], ones)
                pltpu.sync_copy(ones, zref.at[pl.ds(row0 + i * _B, _B)])

        @pl.when(c == 0)
        def _():
            wb(deg0)

        @pl.when(c == 1)
        def _():
            wb(deg1)

    return deg_kernel


def _edge_pipeline(yref, acc, srcv, dstv, rows0, rows1, gs0, gs1,
                   ss0, ss1, nb):
    pltpu.async_copy(yref.at[srcv.at[0]], rows0, gs0)
    pltpu.async_copy(yref.at[srcv.at[1]], rows1, gs1)

    def step(g, _):
        j0 = g * 2
        pltpu.make_async_copy(yref.at[srcv.at[j0]], rows0, gs0).wait()
        s0 = pltpu.async_copy(rows0, acc.at[dstv.at[j0]], ss0, add=True)
        pltpu.make_async_copy(yref.at[srcv.at[j0 + 1]], rows1, gs1).wait()
        s1 = pltpu.async_copy(rows1, acc.at[dstv.at[j0 + 1]], ss1, add=True)
        s0.wait()

        @pl.when(j0 + 2 < nb)
        def _():
            pltpu.async_copy(yref.at[srcv.at[j0 + 2]], rows0, gs0)

        s1.wait()

        @pl.when(j0 + 3 < nb)
        def _():
            pltpu.async_copy(yref.at[srcv.at[j0 + 3]], rows1, gs1)

        return 0

    lax.fori_loop(0, nb // 2, step, 0)


def _prop_scratch():
    return [
        pltpu.VMEM((_NBC, _B), jnp.int32),
        pltpu.VMEM((_NBC, _B), jnp.int32),
        pltpu.VMEM((_B, 128), _f32),
        pltpu.VMEM((_B, 128), _f32),
        pltpu.VMEM_SHARED((_NP, 128), _f32),
        pltpu.SemaphoreType.DMA,
        pltpu.SemaphoreType.DMA,
        pltpu.SemaphoreType.DMA,
        pltpu.SemaphoreType.DMA,
    ]


def _zero_acc(acc, rows0, row0, sem):
    _zero_rows(rows0, _B, 128)
    descs = [
        pltpu.async_copy(rows0, acc.at[pl.ds(row0 + i * _B, _B)], sem)
        for i in range(_RPT // _B)
    ]
    for d in descs:
        d.wait()


def _writeback(acc, rows0, rows1, row0, zref, ss0, ss1):
    bufs = (rows0, rows1)
    sems = (ss0, ss1)
    descs = [None, None]
    for i in range(_RPT // _B):
        b = i % 2
        if descs[b] is not None:
            descs[b].wait()
        pltpu.sync_copy(acc.at[pl.ds(row0 + i * _B, _B)], bufs[b])
        descs[b] = pltpu.async_copy(
            bufs[b], zref.at[pl.ds(row0 + i * _B, _B)], sems[b])
    for d in descs:
        if d is not None:
            d.wait()


def _chunked_pipeline(yref, acc, src_hbm, dst_hbm, srcv, dstv,
                      rows0, rows1, gs0, gs1, ss0, ss1, m0, nchunks):
    for k in range(nchunks):
        pltpu.sync_copy(src_hbm.at[m0 + k], srcv)
        pltpu.sync_copy(dst_hbm.at[m0 + k], dstv)
        _edge_pipeline(yref, acc, srcv, dstv, rows0, rows1, gs0, gs1,
                       ss0, ss1, _NBC)


def _make_prop_edgesplit():
    mesh = plsc.VectorSubcoreMesh(core_axis_name="c", subcore_axis_name="s")

    @functools.partial(
        pl.kernel,
        out_type=(
            jax.ShapeDtypeStruct((_NP, 128), _f32),
            jax.ShapeDtypeStruct((_NP, 128), _f32),
        ),
        mesh=mesh,
        scratch_types=_prop_scratch(),
    )
    def prop_kernel(y, src_hbm, dst_hbm, z0, z1,
                    srcv, dstv, rows0, rows1, acc, gs0, gs1, ss0, ss1):
        c = lax.axis_index("c")
        s = lax.axis_index("s")
        row0 = s * _RPT
        w = c * 16 + s

        _zero_acc(acc, rows0, row0, gs0)
        plsc.subcore_barrier()

        _chunked_pipeline(y, acc, src_hbm, dst_hbm, srcv, dstv,
                          rows0, rows1, gs0, gs1, ss0, ss1, 4 * w, 4)
        plsc.subcore_barrier()

        @pl.when(c == 0)
        def _():
            _writeback(acc, rows0, rows1, row0, z0, ss0, ss1)

        @pl.when(c == 1)
        def _():
            _writeback(acc, rows0, rows1, row0, z1, ss0, ss1)

    return prop_kernel


def _make_prop_dimsplit():
    mesh = plsc.VectorSubcoreMesh(core_axis_name="c", subcore_axis_name="s")

    @functools.partial(
        pl.kernel,
        out_type=(
            jax.ShapeDtypeStruct((_NP, 128), _f32),
            jax.ShapeDtypeStruct((_NP, 128), _f32),
        ),
        mesh=mesh,
        scratch_types=_prop_scratch(),
    )
    def prop_kernel(y0, y1, src_hbm, dst_hbm, z0, z1,
                    srcv, dstv, rows0, rows1, acc, gs0, gs1, ss0, ss1):
        c = lax.axis_index("c")
        s = lax.axis_index("s")
        row0 = s * _RPT

        _zero_acc(acc, rows0, row0, gs0)
        plsc.subcore_barrier()

        def go(yref, zref):
            _chunked_pipeline(yref, acc, src_hbm, dst_hbm, srcv, dstv,
                              rows0, rows1, gs0, gs1, ss0, ss1, 8 * s, 8)
            plsc.subcore_barrier()
            _writeback(acc, rows0, rows1, row0, zref, ss0, ss1)

        @pl.when(c == 0)
        def _():
            go(y0, z0)

        @pl.when(c == 1)
        def _():
            go(y1, z1)

    return prop_kernel


def _norm_scale(deg0, deg1, x):
    def body(d0_ref, d1_ref, x_ref, norm_ref, y_ref):
        deg = d0_ref[:, :1] + d1_ref[:, :1]
        norm = lax.rsqrt(jnp.maximum(deg, 1.0))
        norm_ref[...] = norm
        y_ref[...] = x_ref[...] * norm

    return pl.pallas_call(
        body,
        out_shape=(
            jax.ShapeDtypeStruct((_NP, 1), _f32),
            jax.ShapeDtypeStruct((_NP, 128), _f32),
        ),
    )(deg0, deg1, x)


def _rescale_es(z0, z1, norm):
    def body(z0_ref, z1_ref, n_ref, y_ref):
        n2 = n_ref[...] * n_ref[...]
        y_ref[...] = (z0_ref[...] + z1_ref[...]) * n2

    return pl.pallas_call(
        body,
        out_shape=jax.ShapeDtypeStruct(z0.shape, _f32),
    )(z0, z1, norm)


def _rescale_ds(z0, z1, norm):
    def body(z0_ref, z1_ref, n_ref, y0_ref, y1_ref):
        n2 = n_ref[...] * n_ref[...]
        y0_ref[...] = z0_ref[...] * n2
        y1_ref[...] = z1_ref[...] * n2

    return pl.pallas_call(
        body,
        out_shape=(
            jax.ShapeDtypeStruct(z0.shape, _f32),
            jax.ShapeDtypeStruct(z1.shape, _f32),
        ),
    )(z0, z1, norm)


def _layer0_mm(x, z1p, z2p, norm, W, b):
    BR = 1024

    def body(x_ref, z1a_ref, z1b_ref, z2a_ref, z2b_ref, n_ref, w_ref, b_ref,
             out_ref, y0_ref, y1_ref):
        n = n_ref[...]
        hop1 = (z1a_ref[...] + z1b_ref[...]) * n
        hop2 = (z2a_ref[...] + z2b_ref[...]) * n
        acc = jnp.dot(x_ref[...], w_ref[:128, :], preferred_element_type=_f32)
        acc += jnp.dot(hop1, w_ref[128:256, :], preferred_element_type=_f32)
        acc += jnp.dot(hop2, w_ref[256:, :], preferred_element_type=_f32)
        acc += b_ref[...]
        acc = jnp.maximum(acc, 0.0)
        out_ref[...] = acc
        y = acc * n
        y0_ref[...] = y[:, :128]
        y1_ref[...] = y[:, 128:]

    blk = lambda cols: pl.BlockSpec((BR, cols), lambda i: (i, 0))
    return pl.pallas_call(
        body,
        grid=(_NP // BR,),
        in_specs=[
            blk(128), blk(128), blk(128), blk(128), blk(128), blk(1),
            pl.BlockSpec((384, 256), lambda i: (0, 0)),
            pl.BlockSpec((1, 256), lambda i: (0, 0)),
        ],
        out_specs=(blk(256), blk(128), blk(128)),
        out_shape=(
            jax.ShapeDtypeStruct((_NP, 256), _f32),
            jax.ShapeDtypeStruct((_NP, 128), _f32),
            jax.ShapeDtypeStruct((_NP, 128), _f32),
        ),
    )(x, z1p[0], z1p[1], z2p[0], z2p[1], norm, W, b.reshape(1, 256))


def _layer1_mm(h, z1p, z2p, norm, W, b, Wpq):
    BR = 1024

    def body(h_ref, z1a_ref, z1b_ref, z2a_ref, z2b_ref, n_ref, w_ref, b_ref,
             wpq_ref, out_ref, p_ref, r0_ref):
        n = n_ref[...]
        hop1 = jnp.concatenate([z1a_ref[...], z1b_ref[...]], axis=1) * n
        hop2 = jnp.concatenate([z2a_ref[...], z2b_ref[...]], axis=1) * n
        acc = jnp.dot(h_ref[...], w_ref[:256, :], preferred_element_type=_f32)
        acc += jnp.dot(hop1, w_ref[256:512, :], preferred_element_type=_f32)
        acc += jnp.dot(hop2, w_ref[512:, :], preferred_element_type=_f32)
        acc += b_ref[...]
        acc = jnp.maximum(acc, 0.0)
        out_ref[...] = acc
        proj = jnp.dot(acc, wpq_ref[...], preferred_element_type=_f32)
        p_ref[...] = proj[:, :16]
        r0 = proj[:, 16:] * n
        r0_ref[...] = jnp.concatenate(
            [r0, jnp.zeros((BR, 112), _f32)], axis=1)

    blk = lambda cols: pl.BlockSpec((BR, cols), lambda i: (i, 0))
    return pl.pallas_call(
        body,
        grid=(_NP // BR,),
        in_specs=[
            blk(256), blk(128), blk(128), blk(128), blk(128), blk(1),
            pl.BlockSpec((768, 256), lambda i: (0, 0)),
            pl.BlockSpec((1, 256), lambda i: (0, 0)),
            pl.BlockSpec((256, 32), lambda i: (0, 0)),
        ],
        out_specs=(blk(256), blk(16), blk(128)),
        out_shape=(
            jax.ShapeDtypeStruct((_NP, 256), _f32),
            jax.ShapeDtypeStruct((_NP, 16), _f32),
            jax.ShapeDtypeStruct((_NP, 128), _f32),
        ),
    )(h, z1p[0], z1p[1], z2p[0], z2p[1], norm, W, b.reshape(1, 256), Wpq)


def _mid16(p, t0, t1, norm):
    def body(p_ref, t0_ref, t1_ref, n_ref, r_ref):
        n = n_ref[...]
        t = (t0_ref[:, :16] + t1_ref[:, :16]) * (n * n)
        r = p_ref[...] * n + t
        r_ref[...] = jnp.concatenate(
            [r, jnp.zeros((_NP, 112), _f32)], axis=1)

    return pl.pallas_call(
        body,
        out_shape=jax.ShapeDtypeStruct((_NP, 128), _f32),
    )(p, t0, t1, norm)


def _final(h2, W2a, b2, u0, u1, norm):
    BR = 2048

    def body(h_ref, w_ref, b_ref, u0_ref, u1_ref, n_ref, o_ref):
        acc = jnp.dot(h_ref[...], w_ref[...], preferred_element_type=_f32)
        u = (u0_ref[:, :16] + u1_ref[:, :16]) * n_ref[...]
        o_ref[...] = acc + b_ref[...] + u

    return pl.pallas_call(
        body,
        grid=(_NP // BR,),
        in_specs=[
            pl.BlockSpec((BR, 256), lambda i: (i, 0)),
            pl.BlockSpec((256, 16), lambda i: (0, 0)),
            pl.BlockSpec((1, 16), lambda i: (0, 0)),
            pl.BlockSpec((BR, 128), lambda i: (i, 0)),
            pl.BlockSpec((BR, 128), lambda i: (i, 0)),
            pl.BlockSpec((BR, 1), lambda i: (i, 0)),
        ],
        out_specs=pl.BlockSpec((BR, 16), lambda i: (i, 0)),
        out_shape=jax.ShapeDtypeStruct((_NP, 16), _f32),
    )(h2, W2a, b2.reshape(1, 16), u0, u1, norm)


def kernel(features, edge_index, W0, b0, W1, b1, W2, b2):
    src = jnp.concatenate(
        [edge_index[0], jnp.full((_EP - _E,), _NP - 1, jnp.int32)])
    dst = jnp.concatenate(
        [edge_index[1], jnp.full((_EP - _E,), _NP - 1, jnp.int32)])
    src_es = src.reshape(128, _NBC, _B)
    dst_es = dst.reshape(128, _NBC, _B)
    src_ds = src_es
    dst_ds = dst_es

    x = jnp.pad(features, ((0, _NP - _N), (0, 0)))

    deg0, deg1 = _make_deg_kernel()(dst_es)
    norm, y = _norm_scale(deg0, deg1, x)


    prop_es = _make_prop_edgesplit()
    prop_ds = _make_prop_dimsplit()

    z1a, z1b = prop_es(y, src_es, dst_es)
    yb = _rescale_es(z1a, z1b, norm)
    z2a, z2b = prop_es(yb, src_es, dst_es)
    h1, ya0, ya1 = _layer0_mm(x, (z1a, z1b), (z2a, z2b), norm, W0, b0)

    z10, z11 = prop_ds(ya0, ya1, src_ds, dst_ds)
    yb0, yb1 = _rescale_ds(z10, z11, norm)
    z20, z21 = prop_ds(yb0, yb1, src_ds, dst_ds)
    h2, p, r0 = _layer1_mm(h1, (z10, z11), (z20, z21), norm, W1, b1,
                           jnp.concatenate([W2[256:512], W2[512:768]], axis=1))

    t0, t1 = prop_es(r0, src_es, dst_es)
    r1 = _mid16(p, t0, t1, norm)
    u0, u1 = prop_es(r1, src_es, dst_es)
    out = _final(h2, W2[:256], b2, u0, u1, norm)
    return out[:_N]


# speedup vs baseline: 3.5182x; 1.1239x over previous
import functools

import jax
import jax.numpy as jnp
from jax import lax
from jax.experimental import pallas as pl
from jax.experimental.pallas import tpu as pltpu
from jax.experimental.pallas import tpu_sc as plsc

_N = 10000
_NP = 10240
_E = 320000
_EP = 327680
_B = 128
_NBC = 40
_RPT = _NP // 16

_f32 = jnp.float32


def _zero_rows(ref, nrows, ncols):
    zer = jnp.zeros((16,), _f32)

    def row(i, _):
        for k in range(ncols // 16):
            ref[i, pl.ds(k * 16, 16)] = zer
        return 0

    lax.fori_loop(0, nrows, row, 0, unroll=4)


def _make_deg_kernel():
    mesh = plsc.VectorSubcoreMesh(core_axis_name="c", subcore_axis_name="s")

    @functools.partial(
        pl.kernel,
        out_type=(
            jax.ShapeDtypeStruct((_NP, 128), _f32),
            jax.ShapeDtypeStruct((_NP, 128), _f32),
        ),
        mesh=mesh,
        scratch_types=[
            pltpu.VMEM((_NBC, _B), jnp.int32),
            pltpu.VMEM((_B, 128), _f32),
            pltpu.VMEM_SHARED((_NP, 128), _f32),
            pltpu.SemaphoreType.DMA,
            pltpu.SemaphoreType.DMA,
        ],
    )
    def deg_kernel(dst_hbm, deg0, deg1, dstv, ones, acc, ss0, ss1):
        c = lax.axis_index("c")
        s = lax.axis_index("s")
        row0 = s * _RPT
        w = c * 16 + s

        _zero_acc(acc, ones, row0, ss0)
        one = jnp.ones((16,), _f32)

        def orow(i, _):
            for k in range(8):
                ones[i, pl.ds(k * 16, 16)] = one
            return 0

        lax.fori_loop(0, _B, orow, 0, unroll=4)
        plsc.subcore_barrier()

        def step(g, _):
            j0 = g * 2
            s0 = pltpu.async_copy(ones, acc.at[dstv.at[j0]], ss0, add=True)
            s1 = pltpu.async_copy(ones, acc.at[dstv.at[j0 + 1]], ss1, add=True)
            s0.wait()
            s1.wait()
            return 0

        for k in range(2):
            pltpu.sync_copy(dst_hbm.at[2 * w + k], dstv)
            lax.fori_loop(0, _NBC // 2, step, 0)
        plsc.subcore_barrier()

        def wb(zref):
            for i in range(_RPT // _B):
                pltpu.sync_copy(acc.at[pl.ds(row0 + i * _B, _B)], ones)
                pltpu.sync_copy(ones, zref.at[pl.ds(row0 + i * _B, _B)])

        @pl.when(c == 0)
        def _():
            wb(deg0)

        @pl.when(c == 1)
        def _():
            wb(deg1)

    return deg_kernel


def _edge_pipeline(yref, acc, srcv, dstv, rows0, rows1, gs0, gs1,
                   ss0, ss1, nb):
    pltpu.async_copy(yref.at[srcv.at[0]], rows0, gs0)
    pltpu.async_copy(yref.at[srcv.at[1]], rows1, gs1)

    def step(g, _):
        j0 = g * 2
        pltpu.make_async_copy(yref.at[srcv.at[j0]], rows0, gs0).wait()
        s0 = pltpu.async_copy(rows0, acc.at[dstv.at[j0]], ss0, add=True)
        pltpu.make_async_copy(yref.at[srcv.at[j0 + 1]], rows1, gs1).wait()
        s1 = pltpu.async_copy(rows1, acc.at[dstv.at[j0 + 1]], ss1, add=True)
        s0.wait()

        @pl.when(j0 + 2 < nb)
        def _():
            pltpu.async_copy(yref.at[srcv.at[j0 + 2]], rows0, gs0)

        s1.wait()

        @pl.when(j0 + 3 < nb)
        def _():
            pltpu.async_copy(yref.at[srcv.at[j0 + 3]], rows1, gs1)

        return 0

    lax.fori_loop(0, nb // 2, step, 0)


def _prop_scratch():
    return [
        pltpu.VMEM((_NBC, _B), jnp.int32),
        pltpu.VMEM((_NBC, _B), jnp.int32),
        pltpu.VMEM((_B, 128), _f32),
        pltpu.VMEM((_B, 128), _f32),
        pltpu.VMEM_SHARED((_NP, 128), _f32),
        pltpu.SemaphoreType.DMA,
        pltpu.SemaphoreType.DMA,
        pltpu.SemaphoreType.DMA,
        pltpu.SemaphoreType.DMA,
    ]


def _zero_acc(acc, rows0, row0, sem):
    _zero_rows(rows0, _B, 128)
    descs = [
        pltpu.async_copy(rows0, acc.at[pl.ds(row0 + i * _B, _B)], sem)
        for i in range(_RPT // _B)
    ]
    for d in descs:
        d.wait()


def _writeback(acc, rows0, rows1, row0, zref, ss0, ss1):
    bufs = (rows0, rows1)
    sems = (ss0, ss1)
    descs = [None, None]
    for i in range(_RPT // _B):
        b = i % 2
        if descs[b] is not None:
            descs[b].wait()
        pltpu.sync_copy(acc.at[pl.ds(row0 + i * _B, _B)], bufs[b])
        descs[b] = pltpu.async_copy(
            bufs[b], zref.at[pl.ds(row0 + i * _B, _B)], sems[b])
    for d in descs:
        if d is not None:
            d.wait()


def _chunked_pipeline(yref, acc, src_hbm, dst_hbm, srcv, dstv,
                      rows0, rows1, gs0, gs1, ss0, ss1, m0, nchunks):
    for k in range(nchunks):
        pltpu.sync_copy(src_hbm.at[m0 + k], srcv)
        pltpu.sync_copy(dst_hbm.at[m0 + k], dstv)
        _edge_pipeline(yref, acc, srcv, dstv, rows0, rows1, gs0, gs1,
                       ss0, ss1, _NBC)


def _make_prop_edgesplit():
    mesh = plsc.VectorSubcoreMesh(core_axis_name="c", subcore_axis_name="s")

    @functools.partial(
        pl.kernel,
        out_type=(
            jax.ShapeDtypeStruct((_NP, 128), _f32),
            jax.ShapeDtypeStruct((_NP, 128), _f32),
        ),
        mesh=mesh,
        scratch_types=_prop_scratch(),
    )
    def prop_kernel(y, src_hbm, dst_hbm, z0, z1,
                    srcv, dstv, rows0, rows1, acc, gs0, gs1, ss0, ss1):
        c = lax.axis_index("c")
        s = lax.axis_index("s")
        row0 = s * _RPT
        w = c * 16 + s

        _zero_acc(acc, rows0, row0, gs0)
        plsc.subcore_barrier()

        _chunked_pipeline(y, acc, src_hbm, dst_hbm, srcv, dstv,
                          rows0, rows1, gs0, gs1, ss0, ss1, 2 * w, 2)
        plsc.subcore_barrier()

        @pl.when(c == 0)
        def _():
            _writeback(acc, rows0, rows1, row0, z0, ss0, ss1)

        @pl.when(c == 1)
        def _():
            _writeback(acc, rows0, rows1, row0, z1, ss0, ss1)

    return prop_kernel


def _make_prop_dimsplit():
    mesh = plsc.VectorSubcoreMesh(core_axis_name="c", subcore_axis_name="s")

    @functools.partial(
        pl.kernel,
        out_type=(
            jax.ShapeDtypeStruct((_NP, 128), _f32),
            jax.ShapeDtypeStruct((_NP, 128), _f32),
        ),
        mesh=mesh,
        scratch_types=_prop_scratch(),
    )
    def prop_kernel(y0, y1, src_hbm, dst_hbm, z0, z1,
                    srcv, dstv, rows0, rows1, acc, gs0, gs1, ss0, ss1):
        c = lax.axis_index("c")
        s = lax.axis_index("s")
        row0 = s * _RPT

        _zero_acc(acc, rows0, row0, gs0)
        plsc.subcore_barrier()

        def go(yref, zref):
            _chunked_pipeline(yref, acc, src_hbm, dst_hbm, srcv, dstv,
                              rows0, rows1, gs0, gs1, ss0, ss1, 4 * s, 4)
            plsc.subcore_barrier()
            _writeback(acc, rows0, rows1, row0, zref, ss0, ss1)

        @pl.when(c == 0)
        def _():
            go(y0, z0)

        @pl.when(c == 1)
        def _():
            go(y1, z1)

    return prop_kernel


def _norm_scale(deg0, deg1, x):
    def body(d0_ref, d1_ref, x_ref, norm_ref, y_ref):
        deg = d0_ref[:, :1] + d1_ref[:, :1]
        norm = lax.rsqrt(jnp.maximum(deg, 1.0))
        norm_ref[...] = norm
        y_ref[...] = x_ref[...] * norm

    return pl.pallas_call(
        body,
        out_shape=(
            jax.ShapeDtypeStruct((_NP, 1), _f32),
            jax.ShapeDtypeStruct((_NP, 128), _f32),
        ),
    )(deg0, deg1, x)


def _rescale_es(z0, z1, norm):
    def body(z0_ref, z1_ref, n_ref, y_ref):
        n2 = n_ref[...] * n_ref[...]
        y_ref[...] = (z0_ref[...] + z1_ref[...]) * n2

    return pl.pallas_call(
        body,
        out_shape=jax.ShapeDtypeStruct(z0.shape, _f32),
    )(z0, z1, norm)


def _rescale_ds(z0, z1, norm):
    def body(z0_ref, z1_ref, n_ref, y0_ref, y1_ref):
        n2 = n_ref[...] * n_ref[...]
        y0_ref[...] = z0_ref[...] * n2
        y1_ref[...] = z1_ref[...] * n2

    return pl.pallas_call(
        body,
        out_shape=(
            jax.ShapeDtypeStruct(z0.shape, _f32),
            jax.ShapeDtypeStruct(z1.shape, _f32),
        ),
    )(z0, z1, norm)


def _layer0_mm(x, z1p, z2p, norm, W, b):
    BR = 1024

    def body(x_ref, z1a_ref, z1b_ref, z2a_ref, z2b_ref, n_ref, w_ref, b_ref,
             out_ref, y0_ref, y1_ref):
        n = n_ref[...]
        hop1 = (z1a_ref[...] + z1b_ref[...]) * n
        hop2 = (z2a_ref[...] + z2b_ref[...]) * n
        acc = jnp.dot(x_ref[...], w_ref[:128, :], preferred_element_type=_f32)
        acc += jnp.dot(hop1, w_ref[128:256, :], preferred_element_type=_f32)
        acc += jnp.dot(hop2, w_ref[256:, :], preferred_element_type=_f32)
        acc += b_ref[...]
        acc = jnp.maximum(acc, 0.0)
        out_ref[...] = acc
        y = acc * n
        y0_ref[...] = y[:, :128]
        y1_ref[...] = y[:, 128:]

    blk = lambda cols: pl.BlockSpec((BR, cols), lambda i: (i, 0))
    return pl.pallas_call(
        body,
        grid=(_NP // BR,),
        in_specs=[
            blk(128), blk(128), blk(128), blk(128), blk(128), blk(1),
            pl.BlockSpec((384, 256), lambda i: (0, 0)),
            pl.BlockSpec((1, 256), lambda i: (0, 0)),
        ],
        out_specs=(blk(256), blk(128), blk(128)),
        out_shape=(
            jax.ShapeDtypeStruct((_NP, 256), _f32),
            jax.ShapeDtypeStruct((_NP, 128), _f32),
            jax.ShapeDtypeStruct((_NP, 128), _f32),
        ),
    )(x, z1p[0], z1p[1], z2p[0], z2p[1], norm, W, b.reshape(1, 256))


def _layer1_mm(h, z1p, z2p, norm, W, b, Wpq):
    BR = 1024

    def body(h_ref, z1a_ref, z1b_ref, z2a_ref, z2b_ref, n_ref, w_ref, b_ref,
             wpq_ref, out_ref, p_ref, r0_ref):
        n = n_ref[...]
        hop1 = jnp.concatenate([z1a_ref[...], z1b_ref[...]], axis=1) * n
        hop2 = jnp.concatenate([z2a_ref[...], z2b_ref[...]], axis=1) * n
        acc = jnp.dot(h_ref[...], w_ref[:256, :], preferred_element_type=_f32)
        acc += jnp.dot(hop1, w_ref[256:512, :], preferred_element_type=_f32)
        acc += jnp.dot(hop2, w_ref[512:, :], preferred_element_type=_f32)
        acc += b_ref[...]
        acc = jnp.maximum(acc, 0.0)
        out_ref[...] = acc
        proj = jnp.dot(acc, wpq_ref[...], preferred_element_type=_f32)
        p_ref[...] = proj[:, :16]
        r0 = proj[:, 16:] * n
        r0_ref[...] = jnp.concatenate(
            [r0, jnp.zeros((BR, 112), _f32)], axis=1)

    blk = lambda cols: pl.BlockSpec((BR, cols), lambda i: (i, 0))
    return pl.pallas_call(
        body,
        grid=(_NP // BR,),
        in_specs=[
            blk(256), blk(128), blk(128), blk(128), blk(128), blk(1),
            pl.BlockSpec((768, 256), lambda i: (0, 0)),
            pl.BlockSpec((1, 256), lambda i: (0, 0)),
            pl.BlockSpec((256, 32), lambda i: (0, 0)),
        ],
        out_specs=(blk(256), blk(16), blk(128)),
        out_shape=(
            jax.ShapeDtypeStruct((_NP, 256), _f32),
            jax.ShapeDtypeStruct((_NP, 16), _f32),
            jax.ShapeDtypeStruct((_NP, 128), _f32),
        ),
    )(h, z1p[0], z1p[1], z2p[0], z2p[1], norm, W, b.reshape(1, 256), Wpq)


def _mid16(p, t0, t1, norm):
    def body(p_ref, t0_ref, t1_ref, n_ref, r_ref):
        n = n_ref[...]
        t = (t0_ref[:, :16] + t1_ref[:, :16]) * (n * n)
        r = p_ref[...] * n + t
        r_ref[...] = jnp.concatenate(
            [r, jnp.zeros((_NP, 112), _f32)], axis=1)

    return pl.pallas_call(
        body,
        out_shape=jax.ShapeDtypeStruct((_NP, 128), _f32),
    )(p, t0, t1, norm)


def _final(h2, W2a, b2, u0, u1, norm):
    BR = 2048

    def body(h_ref, w_ref, b_ref, u0_ref, u1_ref, n_ref, o_ref):
        acc = jnp.dot(h_ref[...], w_ref[...], preferred_element_type=_f32)
        u = (u0_ref[:, :16] + u1_ref[:, :16]) * n_ref[...]
        o_ref[...] = acc + b_ref[...] + u

    return pl.pallas_call(
        body,
        grid=(_NP // BR,),
        in_specs=[
            pl.BlockSpec((BR, 256), lambda i: (i, 0)),
            pl.BlockSpec((256, 16), lambda i: (0, 0)),
            pl.BlockSpec((1, 16), lambda i: (0, 0)),
            pl.BlockSpec((BR, 128), lambda i: (i, 0)),
            pl.BlockSpec((BR, 128), lambda i: (i, 0)),
            pl.BlockSpec((BR, 1), lambda i: (i, 0)),
        ],
        out_specs=pl.BlockSpec((BR, 16), lambda i: (i, 0)),
        out_shape=jax.ShapeDtypeStruct((_NP, 16), _f32),
    )(h2, W2a, b2.reshape(1, 16), u0, u1, norm)


def kernel(features, edge_index, W0, b0, W1, b1, W2, b2):
    src = jnp.concatenate(
        [edge_index[0], jnp.full((_EP - _E,), _NP - 1, jnp.int32)])
    dst = jnp.concatenate(
        [edge_index[1], jnp.full((_EP - _E,), _NP - 1, jnp.int32)])
    src_es = src.reshape(64, _NBC, _B)
    dst_es = dst.reshape(64, _NBC, _B)
    src_ds = src_es
    dst_ds = dst_es

    x = jnp.pad(features, ((0, _NP - _N), (0, 0)))

    deg0, deg1 = _make_deg_kernel()(dst_es)
    norm, y = _norm_scale(deg0, deg1, x)


    prop_es = _make_prop_edgesplit()
    prop_ds = _make_prop_dimsplit()

    z1a, z1b = prop_es(y, src_es, dst_es)
    yb = _rescale_es(z1a, z1b, norm)
    z2a, z2b = prop_es(yb, src_es, dst_es)
    h1, ya0, ya1 = _layer0_mm(x, (z1a, z1b), (z2a, z2b), norm, W0, b0)

    z10, z11 = prop_ds(ya0, ya1, src_ds, dst_ds)
    yb0, yb1 = _rescale_ds(z10, z11, norm)
    z20, z21 = prop_ds(yb0, yb1, src_ds, dst_ds)
    h2, p, r0 = _layer1_mm(h1, (z10, z11), (z20, z21), norm, W1, b1,
                           jnp.concatenate([W2[256:512], W2[512:768]], axis=1))

    t0, t1 = prop_es(r0, src_es, dst_es)
    r1 = _mid16(p, t0, t1, norm)
    u0, u1 = prop_es(r1, src_es, dst_es)
    out = _final(h2, W2[:256], b2, u0, u1, norm)
    return out[:_N]
